# Initial kernel scaffold; baseline (speedup 1.0000x reference)
import jax
import jax.numpy as jnp
from jax.experimental import pallas as pl


def kernel(node_feature, edge_index, W_src1, b_src1, W_dst1, b_dst1, W_upd1, b_upd1, W_src2, b_src2, W_dst2, b_dst2, W_upd2, b_upd2, gamma1, beta1, gamma2, beta2, W_post, b_post):
    raise NotImplementedError("write your pallas kernel here")


# speedup vs baseline: 6.7466x; 6.7466x over previous
import functools
import jax
import jax.numpy as jnp
from jax import lax
from jax.experimental import pallas as pl
from jax.experimental.pallas import tpu as pltpu
from jax.experimental.pallas import tpu_sc as plsc

N = 10000
E = 320000
D = 128
H = 128
L = 16

NC = 2
NS = 16
NW = NC * NS
PT = E // NW
CH = 80
NCH = PT // CH
STRIPE = 632
LAST = N - STRIPE * (NS - 1)


def _sc_zero_stripe(sid, ref, zf_hbm, off):
  @pl.when(sid < NS - 1)
  def _():
    pltpu.sync_copy(zf_hbm, ref.at[pl.ds(off, STRIPE)])

  @pl.when(sid == NS - 1)
  def _():
    pltpu.sync_copy(zf_hbm.at[pl.ds(0, LAST)], ref.at[pl.ds(off, LAST)])


def _sc_write_stripe(sid, cid, ref, out, off):
  @pl.when(sid < NS - 1)
  def _():
    pltpu.sync_copy(ref.at[pl.ds(off, STRIPE)], out.at[cid, pl.ds(off, STRIPE)])

  @pl.when(sid == NS - 1)
  def _():
    pltpu.sync_copy(ref.at[pl.ds(off, LAST)], out.at[cid, pl.ds(off, LAST)])


@functools.lru_cache(None)
def _make_sc_aggregate():
  mesh = plsc.VectorSubcoreMesh(core_axis_name="c", subcore_axis_name="s")

  def body(x_hbm, ei_hbm, zf_hbm, feat_out, acc, ebuf_a, ebuf_b,
           rbuf_a, rbuf_b, isem_a, isem_b, sem_a, sem_b):
    cid = lax.axis_index("c")
    sid = lax.axis_index("s")
    w = cid * NS + sid
    off = pl.multiple_of(sid * STRIPE, 8)

    _sc_zero_stripe(sid, acc, zf_hbm, off)
    plsc.subcore_barrier()

    def chunk_pair(i, carry):
      j = 2 * i
      hi_a = pltpu.async_copy(ei_hbm.at[w, j], ebuf_a, isem_a)
      hi_b = pltpu.async_copy(ei_hbm.at[w, j + 1], ebuf_b, isem_b)
      hi_a.wait()
      hg_a = pltpu.async_copy(x_hbm.at[ebuf_a.at[0]], rbuf_a, sem_a)
      hi_b.wait()
      hg_b = pltpu.async_copy(x_hbm.at[ebuf_b.at[0]], rbuf_b, sem_b)
      hg_a.wait()
      pltpu.sync_copy(rbuf_a, acc.at[ebuf_a.at[1]], add=True)
      hg_b.wait()
      pltpu.sync_copy(rbuf_b, acc.at[ebuf_b.at[1]], add=True)
      return carry

    lax.fori_loop(0, NCH // 2, chunk_pair, 0)
    pltpu.async_copy(ei_hbm.at[w, NCH - 1], ebuf_a, isem_a).wait()
    pltpu.async_copy(x_hbm.at[ebuf_a.at[0]], rbuf_a, sem_a).wait()
    pltpu.sync_copy(rbuf_a, acc.at[ebuf_a.at[1]], add=True)

    plsc.subcore_barrier()
    _sc_write_stripe(sid, cid, acc, feat_out, off)

  return pl.kernel(
      body,
      out_type=jax.ShapeDtypeStruct((NC, N, D), jnp.float32),
      mesh=mesh,
      scratch_types=(
          pltpu.VMEM_SHARED((N, D), jnp.float32),
          pltpu.VMEM((2, CH), jnp.int32),
          pltpu.VMEM((2, CH), jnp.int32),
          pltpu.VMEM((CH, D), jnp.float32),
          pltpu.VMEM((CH, D), jnp.float32),
          pltpu.SemaphoreType.DMA,
          pltpu.SemaphoreType.DMA,
          pltpu.SemaphoreType.DMA,
          pltpu.SemaphoreType.DMA,
      ))


@functools.lru_cache(None)
def _make_sc_degree():
  mesh = plsc.VectorSubcoreMesh(core_axis_name="c", subcore_axis_name="s")

  def body(ei_hbm, zf_hbm, ones_hbm, deg_out, dacc, ebuf_a, ebuf_b,
           rones, isem_a, isem_b, osem):
    cid = lax.axis_index("c")
    sid = lax.axis_index("s")
    w = cid * NS + sid
    off = pl.multiple_of(sid * STRIPE, 8)

    _sc_zero_stripe(sid, dacc, zf_hbm, off)
    pltpu.async_copy(ones_hbm, rones, osem).wait()
    plsc.subcore_barrier()

    def chunk_pair(i, carry):
      j = 2 * i
      hi_a = pltpu.async_copy(ei_hbm.at[w, j], ebuf_a, isem_a)
      hi_b = pltpu.async_copy(ei_hbm.at[w, j + 1], ebuf_b, isem_b)
      hi_a.wait()
      pltpu.sync_copy(rones, dacc.at[ebuf_a.at[1]], add=True)
      hi_b.wait()
      pltpu.sync_copy(rones, dacc.at[ebuf_b.at[1]], add=True)
      return carry

    lax.fori_loop(0, NCH // 2, chunk_pair, 0)
    pltpu.async_copy(ei_hbm.at[w, NCH - 1], ebuf_a, isem_a).wait()
    pltpu.sync_copy(rones, dacc.at[ebuf_a.at[1]], add=True)

    plsc.subcore_barrier()
    _sc_write_stripe(sid, cid, dacc, deg_out, off)

  return pl.kernel(
      body,
      out_type=jax.ShapeDtypeStruct((NC, N, D), jnp.float32),
      mesh=mesh,
      scratch_types=(
          pltpu.VMEM_SHARED((N, D), jnp.float32),
          pltpu.VMEM((2, CH), jnp.int32),
          pltpu.VMEM((2, CH), jnp.int32),
          pltpu.VMEM((CH, D), jnp.float32),
          pltpu.SemaphoreType.DMA,
          pltpu.SemaphoreType.DMA,
          pltpu.SemaphoreType.DMA,
      ))


BN = 1000
GRID = N // BN


def _tc_pre_body(x_ref, p_ref, deg_ref, wdst_ref, wsrc_ref, wut_ref, wub_ref,
                 bdst_ref, bsrc_ref, bupd_ref, pre_ref, stats_ref):
  s = p_ref[0] + p_ref[1]
  d = deg_ref[0, :, 0:1] + deg_ref[1, :, 0:1]
  mean = s * (1.0 / jnp.maximum(d, 1.0))
  h_dst = jnp.dot(x_ref[...], wdst_ref[...], preferred_element_type=jnp.float32)
  h_src = jnp.dot(mean, wsrc_ref[...], preferred_element_type=jnp.float32)
  bias = (jnp.dot(bdst_ref[...], wut_ref[...], preferred_element_type=jnp.float32)
          + jnp.dot(bsrc_ref[...], wub_ref[...], preferred_element_type=jnp.float32)
          + bupd_ref[...])
  pre = (jnp.dot(h_dst, wut_ref[...], preferred_element_type=jnp.float32)
         + jnp.dot(h_src, wub_ref[...], preferred_element_type=jnp.float32)
         + bias)
  pre_ref[...] = pre

  @pl.when(pl.program_id(0) == 0)
  def _():
    stats_ref[...] = jnp.zeros_like(stats_ref)

  stats_ref[0:1, :] += jnp.sum(pre, axis=0, keepdims=True)
  stats_ref[1:2, :] += jnp.sum(pre * pre, axis=0, keepdims=True)


def _tc_pre(x, parts, degs, w_dst, w_src, w_upd, b_dst, b_src, b_upd):
  wut = w_upd[:H]
  wub = w_upd[H:]
  return pl.pallas_call(
      _tc_pre_body,
      grid=(GRID,),
      in_specs=[
          pl.BlockSpec((BN, D), lambda i: (i, 0)),
          pl.BlockSpec((NC, BN, H), lambda i: (0, i, 0)),
          pl.BlockSpec((NC, BN, D), lambda i: (0, i, 0)),
          pl.BlockSpec((D, H), lambda i: (0, 0)),
          pl.BlockSpec((H, H), lambda i: (0, 0)),
          pl.BlockSpec((H, H), lambda i: (0, 0)),
          pl.BlockSpec((H, H), lambda i: (0, 0)),
          pl.BlockSpec((1, H), lambda i: (0, 0)),
          pl.BlockSpec((1, H), lambda i: (0, 0)),
          pl.BlockSpec((1, H), lambda i: (0, 0)),
      ],
      out_specs=[
          pl.BlockSpec((BN, H), lambda i: (i, 0)),
          pl.BlockSpec((8, H), lambda i: (0, 0)),
      ],
      out_shape=[
          jax.ShapeDtypeStruct((N, H), jnp.float32),
          jax.ShapeDtypeStruct((8, H), jnp.float32),
      ],
  )(x, parts, degs, w_dst, w_src, wut, wub,
    b_dst.reshape(1, H), b_src.reshape(1, H), b_upd.reshape(1, H))


def _tc_norm_body(pre_ref, stats_ref, g_ref, b_ref, out_ref):
  m = stats_ref[0:1, :] / N
  v = stats_ref[1:2, :] / N - m * m
  scale = g_ref[...] / jnp.sqrt(v + 1.0)
  y = (pre_ref[...] - m) * scale + b_ref[...]
  out_ref[...] = jnp.where(y >= 0, y, 0.01 * y)


def _tc_norm(pre, stats, gamma, beta):
  return pl.pallas_call(
      _tc_norm_body,
      grid=(GRID,),
      in_specs=[
          pl.BlockSpec((BN, H), lambda i: (i, 0)),
          pl.BlockSpec((8, H), lambda i: (0, 0)),
          pl.BlockSpec((1, H), lambda i: (0, 0)),
          pl.BlockSpec((1, H), lambda i: (0, 0)),
      ],
      out_specs=pl.BlockSpec((BN, H), lambda i: (i, 0)),
      out_shape=jax.ShapeDtypeStruct((N, H), jnp.float32),
  )(pre, stats, gamma.reshape(1, H), beta.reshape(1, H))


def _tc_norm_head_body(pre_ref, stats_ref, g_ref, b_ref, wp_ref, bp_ref,
                       out_ref):
  m = stats_ref[0:1, :] / N
  v = stats_ref[1:2, :] / N - m * m
  scale = g_ref[...] / jnp.sqrt(v + 1.0)
  y = (pre_ref[...] - m) * scale + b_ref[...]
  y = jnp.where(y >= 0, y, 0.01 * y)
  out_ref[...] = (jnp.dot(y, wp_ref[...], preferred_element_type=jnp.float32)
                  + bp_ref[...])


def _tc_norm_head(pre, stats, gamma, beta, w_post, b_post):
  return pl.pallas_call(
      _tc_norm_head_body,
      grid=(GRID,),
      in_specs=[
          pl.BlockSpec((BN, H), lambda i: (i, 0)),
          pl.BlockSpec((8, H), lambda i: (0, 0)),
          pl.BlockSpec((1, H), lambda i: (0, 0)),
          pl.BlockSpec((1, H), lambda i: (0, 0)),
          pl.BlockSpec((H, L), lambda i: (0, 0)),
          pl.BlockSpec((1, L), lambda i: (0, 0)),
      ],
      out_specs=pl.BlockSpec((BN, L), lambda i: (i, 0)),
      out_shape=jax.ShapeDtypeStruct((N, L), jnp.float32),
  )(pre, stats, gamma.reshape(1, H), beta.reshape(1, H), w_post,
    b_post.reshape(1, L))


@jax.jit
def kernel(node_feature, edge_index, W_src1, b_src1, W_dst1, b_dst1, W_upd1,
           b_upd1, W_src2, b_src2, W_dst2, b_dst2, W_upd2, b_upd2,
           gamma1, beta1, gamma2, beta2, W_post, b_post):
  ei = (edge_index.astype(jnp.int32)
        .reshape(2, NW, NCH, CH).transpose(1, 2, 0, 3))
  zf = jnp.zeros((STRIPE, D), jnp.float32)
  ones = jnp.ones((CH, D), jnp.float32)

  degs = _make_sc_degree()(ei, zf, ones)
  parts1 = _make_sc_aggregate()(node_feature, ei, zf)
  pre1, stats1 = _tc_pre(node_feature, parts1, degs, W_dst1, W_src1, W_upd1,
                         b_dst1, b_src1, b_upd1)
  x1 = _tc_norm(pre1, stats1, gamma1, beta1)

  parts2 = _make_sc_aggregate()(x1, ei, zf)
  pre2, stats2 = _tc_pre(x1, parts2, degs, W_dst2, W_src2, W_upd2,
                         b_dst2, b_src2, b_upd2)
  return _tc_norm_head(pre2, stats2, gamma2, beta2, W_post, b_post)


# speedup vs baseline: 9.8485x; 1.4598x over previous
import functools
import jax
import jax.numpy as jnp
from jax import lax
from jax.experimental import pallas as pl
from jax.experimental.pallas import tpu as pltpu
from jax.experimental.pallas import tpu_sc as plsc

N = 10000
E = 320000
D = 128
H = 128
L = 16

NC = 2
NS = 16
NW = NC * NS
PT = E // NW
CH = 80
NCH = PT // CH
STRIPE = 632
LAST = N - STRIPE * (NS - 1)


def _sc_zero_stripe(sid, ref, zf_hbm, off):
  @pl.when(sid < NS - 1)
  def _():
    pltpu.sync_copy(zf_hbm, ref.at[pl.ds(off, STRIPE)])

  @pl.when(sid == NS - 1)
  def _():
    pltpu.sync_copy(zf_hbm.at[pl.ds(0, LAST)], ref.at[pl.ds(off, LAST)])


def _sc_write_stripe(sid, cid, ref, out, off):
  @pl.when(sid < NS - 1)
  def _():
    pltpu.sync_copy(ref.at[pl.ds(off, STRIPE)], out.at[cid, pl.ds(off, STRIPE)])

  @pl.when(sid == NS - 1)
  def _():
    pltpu.sync_copy(ref.at[pl.ds(off, LAST)], out.at[cid, pl.ds(off, LAST)])


@functools.lru_cache(None)
def _make_sc_aggregate():
  mesh = plsc.VectorSubcoreMesh(core_axis_name="c", subcore_axis_name="s")

  def body(x_hbm, ei_hbm, zf_hbm, feat_out, acc,
           e0, e1, e2, e3, r0, r1, r2, r3,
           is0, is1, is2, is3, gs0, gs1, gs2, gs3, ss0, ss1, ss2, ss3):
    ebuf = (e0, e1, e2, e3)
    rbuf = (r0, r1, r2, r3)
    isem = (is0, is1, is2, is3)
    gsem = (gs0, gs1, gs2, gs3)
    ssem = (ss0, ss1, ss2, ss3)
    cid = lax.axis_index("c")
    sid = lax.axis_index("s")
    w = cid * NS + sid
    off = pl.multiple_of(sid * STRIPE, 8)

    _sc_zero_stripe(sid, acc, zf_hbm, off)
    plsc.subcore_barrier()

    def fetch(j, s):
      pltpu.async_copy(ei_hbm.at[w, j], ebuf[s], isem[s])

    def fetch_wait(s):
      pltpu.make_async_copy(ei_hbm.at[w, 0], ebuf[s], isem[s]).wait()

    def gather(s):
      pltpu.async_copy(x_hbm.at[ebuf[s].at[0]], rbuf[s], gsem[s])

    def gather_wait(s):
      pltpu.make_async_copy(x_hbm.at[ebuf[s].at[0]], rbuf[s], gsem[s]).wait()

    def scat(s):
      pltpu.async_copy(rbuf[s], acc.at[ebuf[s].at[1]], ssem[s], add=True)

    def scat_wait(s):
      pltpu.make_async_copy(rbuf[s], acc.at[ebuf[s].at[1]], ssem[s]).wait()

    fetch(0, 0)
    fetch(1, 1)
    fetch_wait(0)
    gather(0)
    fetch_wait(1)
    gather(1)
    fetch(2, 2)
    fetch_wait(2)
    gather(2)
    gather_wait(0)
    scat(0)
    fetch(3, 3)
    fetch_wait(3)
    gather(3)
    gather_wait(1)
    scat(1)

    def quad(i, carry):
      j0 = 4 * i + 2
      for t in range(4):
        j = j0 + t
        s = (2 + t) % 4
        nxt = (s + 2) % 4
        scat_wait(nxt)
        fetch(j + 2, nxt)
        fetch_wait(nxt)
        gather(nxt)
        gather_wait(s)
        scat(s)
      return carry

    lax.fori_loop(0, (NCH - 5) // 4, quad, 0)

    scat_wait(0)
    fetch(NCH - 1, 0)
    fetch_wait(0)
    gather(0)
    gather_wait(2)
    scat(2)
    scat_wait(1)
    gather_wait(3)
    scat(3)
    gather_wait(0)
    scat(0)
    scat_wait(2)
    scat_wait(3)
    scat_wait(0)

    plsc.subcore_barrier()
    _sc_write_stripe(sid, cid, acc, feat_out, off)

  return pl.kernel(
      body,
      out_type=jax.ShapeDtypeStruct((NC, N, D), jnp.float32),
      mesh=mesh,
      scratch_types=(
          pltpu.VMEM_SHARED((N, D), jnp.float32),
          pltpu.VMEM((2, CH), jnp.int32),
          pltpu.VMEM((2, CH), jnp.int32),
          pltpu.VMEM((2, CH), jnp.int32),
          pltpu.VMEM((2, CH), jnp.int32),
          pltpu.VMEM((CH, D), jnp.float32),
          pltpu.VMEM((CH, D), jnp.float32),
          pltpu.VMEM((CH, D), jnp.float32),
          pltpu.VMEM((CH, D), jnp.float32),
      ) + (pltpu.SemaphoreType.DMA,) * 12)


@functools.lru_cache(None)
def _make_sc_degree():
  mesh = plsc.VectorSubcoreMesh(core_axis_name="c", subcore_axis_name="s")

  def body(ei_hbm, zf_hbm, ones_hbm, deg_out, dacc, ebuf_a, ebuf_b,
           rones, isem_a, isem_b, osem):
    cid = lax.axis_index("c")
    sid = lax.axis_index("s")
    w = cid * NS + sid
    off = pl.multiple_of(sid * STRIPE, 8)

    _sc_zero_stripe(sid, dacc, zf_hbm, off)
    pltpu.async_copy(ones_hbm, rones, osem).wait()
    plsc.subcore_barrier()

    def chunk_pair(i, carry):
      j = 2 * i
      hi_a = pltpu.async_copy(ei_hbm.at[w, j], ebuf_a, isem_a)
      hi_b = pltpu.async_copy(ei_hbm.at[w, j + 1], ebuf_b, isem_b)
      hi_a.wait()
      pltpu.sync_copy(rones, dacc.at[ebuf_a.at[1]], add=True)
      hi_b.wait()
      pltpu.sync_copy(rones, dacc.at[ebuf_b.at[1]], add=True)
      return carry

    lax.fori_loop(0, NCH // 2, chunk_pair, 0)
    pltpu.async_copy(ei_hbm.at[w, NCH - 1], ebuf_a, isem_a).wait()
    pltpu.sync_copy(rones, dacc.at[ebuf_a.at[1]], add=True)

    plsc.subcore_barrier()
    _sc_write_stripe(sid, cid, dacc, deg_out, off)

  return pl.kernel(
      body,
      out_type=jax.ShapeDtypeStruct((NC, N, D), jnp.float32),
      mesh=mesh,
      scratch_types=(
          pltpu.VMEM_SHARED((N, D), jnp.float32),
          pltpu.VMEM((2, CH), jnp.int32),
          pltpu.VMEM((2, CH), jnp.int32),
          pltpu.VMEM((CH, D), jnp.float32),
          pltpu.SemaphoreType.DMA,
          pltpu.SemaphoreType.DMA,
          pltpu.SemaphoreType.DMA,
      ))


BN = 1000
GRID = N // BN


def _tc_pre_body(x_ref, p_ref, deg_ref, wdst_ref, wsrc_ref, wut_ref, wub_ref,
                 bdst_ref, bsrc_ref, bupd_ref, pre_ref, stats_ref):
  s = p_ref[0] + p_ref[1]
  d = deg_ref[0, :, 0:1] + deg_ref[1, :, 0:1]
  mean = s * (1.0 / jnp.maximum(d, 1.0))
  h_dst = jnp.dot(x_ref[...], wdst_ref[...], preferred_element_type=jnp.float32)
  h_src = jnp.dot(mean, wsrc_ref[...], preferred_element_type=jnp.float32)
  bias = (jnp.dot(bdst_ref[...], wut_ref[...], preferred_element_type=jnp.float32)
          + jnp.dot(bsrc_ref[...], wub_ref[...], preferred_element_type=jnp.float32)
          + bupd_ref[...])
  pre = (jnp.dot(h_dst, wut_ref[...], preferred_element_type=jnp.float32)
         + jnp.dot(h_src, wub_ref[...], preferred_element_type=jnp.float32)
         + bias)
  pre_ref[...] = pre

  @pl.when(pl.program_id(0) == 0)
  def _():
    stats_ref[...] = jnp.zeros_like(stats_ref)

  stats_ref[0:1, :] += jnp.sum(pre, axis=0, keepdims=True)
  stats_ref[1:2, :] += jnp.sum(pre * pre, axis=0, keepdims=True)


def _tc_pre(x, parts, degs, w_dst, w_src, w_upd, b_dst, b_src, b_upd):
  wut = w_upd[:H]
  wub = w_upd[H:]
  return pl.pallas_call(
      _tc_pre_body,
      grid=(GRID,),
      in_specs=[
          pl.BlockSpec((BN, D), lambda i: (i, 0)),
          pl.BlockSpec((NC, BN, H), lambda i: (0, i, 0)),
          pl.BlockSpec((NC, BN, D), lambda i: (0, i, 0)),
          pl.BlockSpec((D, H), lambda i: (0, 0)),
          pl.BlockSpec((H, H), lambda i: (0, 0)),
          pl.BlockSpec((H, H), lambda i: (0, 0)),
          pl.BlockSpec((H, H), lambda i: (0, 0)),
          pl.BlockSpec((1, H), lambda i: (0, 0)),
          pl.BlockSpec((1, H), lambda i: (0, 0)),
          pl.BlockSpec((1, H), lambda i: (0, 0)),
      ],
      out_specs=[
          pl.BlockSpec((BN, H), lambda i: (i, 0)),
          pl.BlockSpec((8, H), lambda i: (0, 0)),
      ],
      out_shape=[
          jax.ShapeDtypeStruct((N, H), jnp.float32),
          jax.ShapeDtypeStruct((8, H), jnp.float32),
      ],
  )(x, parts, degs, w_dst, w_src, wut, wub,
    b_dst.reshape(1, H), b_src.reshape(1, H), b_upd.reshape(1, H))


def _tc_norm_body(pre_ref, stats_ref, g_ref, b_ref, out_ref):
  m = stats_ref[0:1, :] / N
  v = stats_ref[1:2, :] / N - m * m
  scale = g_ref[...] / jnp.sqrt(v + 1.0)
  y = (pre_ref[...] - m) * scale + b_ref[...]
  out_ref[...] = jnp.where(y >= 0, y, 0.01 * y)


def _tc_norm(pre, stats, gamma, beta):
  return pl.pallas_call(
      _tc_norm_body,
      grid=(GRID,),
      in_specs=[
          pl.BlockSpec((BN, H), lambda i: (i, 0)),
          pl.BlockSpec((8, H), lambda i: (0, 0)),
          pl.BlockSpec((1, H), lambda i: (0, 0)),
          pl.BlockSpec((1, H), lambda i: (0, 0)),
      ],
      out_specs=pl.BlockSpec((BN, H), lambda i: (i, 0)),
      out_shape=jax.ShapeDtypeStruct((N, H), jnp.float32),
  )(pre, stats, gamma.reshape(1, H), beta.reshape(1, H))


def _tc_norm_head_body(pre_ref, stats_ref, g_ref, b_ref, wp_ref, bp_ref,
                       out_ref):
  m = stats_ref[0:1, :] / N
  v = stats_ref[1:2, :] / N - m * m
  scale = g_ref[...] / jnp.sqrt(v + 1.0)
  y = (pre_ref[...] - m) * scale + b_ref[...]
  y = jnp.where(y >= 0, y, 0.01 * y)
  out_ref[...] = (jnp.dot(y, wp_ref[...], preferred_element_type=jnp.float32)
                  + bp_ref[...])


def _tc_norm_head(pre, stats, gamma, beta, w_post, b_post):
  return pl.pallas_call(
      _tc_norm_head_body,
      grid=(GRID,),
      in_specs=[
          pl.BlockSpec((BN, H), lambda i: (i, 0)),
          pl.BlockSpec((8, H), lambda i: (0, 0)),
          pl.BlockSpec((1, H), lambda i: (0, 0)),
          pl.BlockSpec((1, H), lambda i: (0, 0)),
          pl.BlockSpec((H, L), lambda i: (0, 0)),
          pl.BlockSpec((1, L), lambda i: (0, 0)),
      ],
      out_specs=pl.BlockSpec((BN, L), lambda i: (i, 0)),
      out_shape=jax.ShapeDtypeStruct((N, L), jnp.float32),
  )(pre, stats, gamma.reshape(1, H), beta.reshape(1, H), w_post,
    b_post.reshape(1, L))


@jax.jit
def kernel(node_feature, edge_index, W_src1, b_src1, W_dst1, b_dst1, W_upd1,
           b_upd1, W_src2, b_src2, W_dst2, b_dst2, W_upd2, b_upd2,
           gamma1, beta1, gamma2, beta2, W_post, b_post):
  ei = (edge_index.astype(jnp.int32)
        .reshape(2, NW, NCH, CH).transpose(1, 2, 0, 3))
  zf = jnp.zeros((STRIPE, D), jnp.float32)
  ones = jnp.ones((CH, D), jnp.float32)

  degs = _make_sc_degree()(ei, zf, ones)
  parts1 = _make_sc_aggregate()(node_feature, ei, zf)
  pre1, stats1 = _tc_pre(node_feature, parts1, degs, W_dst1, W_src1, W_upd1,
                         b_dst1, b_src1, b_upd1)
  x1 = _tc_norm(pre1, stats1, gamma1, beta1)

  parts2 = _make_sc_aggregate()(x1, ei, zf)
  pre2, stats2 = _tc_pre(x1, parts2, degs, W_dst2, W_src2, W_upd2,
                         b_dst2, b_src2, b_upd2)
  return _tc_norm_head(pre2, stats2, gamma2, beta2, W_post, b_post)


# speedup vs baseline: 10.7063x; 1.0871x over previous
import functools
import jax
import jax.numpy as jnp
from jax import lax
from jax.experimental import pallas as pl
from jax.experimental.pallas import tpu as pltpu
from jax.experimental.pallas import tpu_sc as plsc

N = 10000
E = 320000
D = 128
H = 128
L = 16

NC = 2
NS = 16
NW = NC * NS
PT = E // NW
CH = 80
NCH = PT // CH
STRIPE = 632
LAST = N - STRIPE * (NS - 1)


def _sc_zero_stripe(sid, ref, zf_hbm, off):
  @pl.when(sid < NS - 1)
  def _():
    pltpu.sync_copy(zf_hbm, ref.at[pl.ds(off, STRIPE)])

  @pl.when(sid == NS - 1)
  def _():
    pltpu.sync_copy(zf_hbm.at[pl.ds(0, LAST)], ref.at[pl.ds(off, LAST)])


def _sc_write_stripe(sid, cid, ref, out, off):
  @pl.when(sid < NS - 1)
  def _():
    pltpu.sync_copy(ref.at[pl.ds(off, STRIPE)], out.at[cid, pl.ds(off, STRIPE)])

  @pl.when(sid == NS - 1)
  def _():
    pltpu.sync_copy(ref.at[pl.ds(off, LAST)], out.at[cid, pl.ds(off, LAST)])


@functools.lru_cache(None)
def _make_sc_aggregate():
  mesh = plsc.VectorSubcoreMesh(core_axis_name="c", subcore_axis_name="s")

  def body(x_hbm, ei_hbm, zf_hbm, feat_out, acc,
           e0, e1, e2, e3, r0, r1, r2, r3,
           is0, is1, is2, is3, gs0, gs1, gs2, gs3, ss0, ss1, ss2, ss3):
    ebuf = (e0, e1, e2, e3)
    rbuf = (r0, r1, r2, r3)
    isem = (is0, is1, is2, is3)
    gsem = (gs0, gs1, gs2, gs3)
    ssem = (ss0, ss1, ss2, ss3)
    cid = lax.axis_index("c")
    sid = lax.axis_index("s")
    w = cid * NS + sid
    off = pl.multiple_of(sid * STRIPE, 8)

    _sc_zero_stripe(sid, acc, zf_hbm, off)
    plsc.subcore_barrier()

    def fetch(j, s):
      pltpu.async_copy(ei_hbm.at[w, j], ebuf[s], isem[s])

    def fetch_wait(s):
      pltpu.make_async_copy(ei_hbm.at[w, 0], ebuf[s], isem[s]).wait()

    def gather(s):
      pltpu.async_copy(x_hbm.at[ebuf[s].at[0]], rbuf[s], gsem[s])

    def gather_wait(s):
      pltpu.make_async_copy(x_hbm.at[ebuf[s].at[0]], rbuf[s], gsem[s]).wait()

    def scat(s):
      pltpu.async_copy(rbuf[s], acc.at[ebuf[s].at[1]], ssem[s], add=True)

    def scat_wait(s):
      pltpu.make_async_copy(rbuf[s], acc.at[ebuf[s].at[1]], ssem[s]).wait()

    fetch(0, 0)
    fetch(1, 1)
    fetch_wait(0)
    gather(0)
    fetch_wait(1)
    gather(1)
    fetch(2, 2)
    fetch_wait(2)
    gather(2)
    gather_wait(0)
    scat(0)
    fetch(3, 3)
    fetch_wait(3)
    gather(3)
    gather_wait(1)
    scat(1)

    def quad(i, carry):
      j0 = 4 * i + 2
      for t in range(4):
        j = j0 + t
        s = (2 + t) % 4
        nxt = (s + 2) % 4
        scat_wait(nxt)
        fetch(j + 2, nxt)
        fetch_wait(nxt)
        gather(nxt)
        gather_wait(s)
        scat(s)
      return carry

    lax.fori_loop(0, (NCH - 5) // 4, quad, 0)

    scat_wait(0)
    fetch(NCH - 1, 0)
    fetch_wait(0)
    gather(0)
    gather_wait(2)
    scat(2)
    scat_wait(1)
    gather_wait(3)
    scat(3)
    gather_wait(0)
    scat(0)
    scat_wait(2)
    scat_wait(3)
    scat_wait(0)

    plsc.subcore_barrier()
    _sc_write_stripe(sid, cid, acc, feat_out, off)

  return pl.kernel(
      body,
      out_type=jax.ShapeDtypeStruct((NC, N, D), jnp.float32),
      mesh=mesh,
      scratch_types=(
          pltpu.VMEM_SHARED((N, D), jnp.float32),
          pltpu.VMEM((2, CH), jnp.int32),
          pltpu.VMEM((2, CH), jnp.int32),
          pltpu.VMEM((2, CH), jnp.int32),
          pltpu.VMEM((2, CH), jnp.int32),
          pltpu.VMEM((CH, D), jnp.float32),
          pltpu.VMEM((CH, D), jnp.float32),
          pltpu.VMEM((CH, D), jnp.float32),
          pltpu.VMEM((CH, D), jnp.float32),
      ) + (pltpu.SemaphoreType.DMA,) * 12)


@functools.lru_cache(None)
def _make_sc_degree():
  mesh = plsc.VectorSubcoreMesh(core_axis_name="c", subcore_axis_name="s")

  def body(ei_hbm, zf_hbm, ones_hbm, deg_out,
           dacc, e0, e1, e2, e3, rones,
           is0, is1, is2, is3, ss0, ss1, ss2, ss3, osem):
    ebuf = (e0, e1, e2, e3)
    isem = (is0, is1, is2, is3)
    ssem = (ss0, ss1, ss2, ss3)
    cid = lax.axis_index("c")
    sid = lax.axis_index("s")
    w = cid * NS + sid
    off = pl.multiple_of(sid * STRIPE, 8)

    _sc_zero_stripe(sid, dacc, zf_hbm, off)
    pltpu.async_copy(ones_hbm, rones, osem).wait()
    plsc.subcore_barrier()

    def fetch(j, s):
      pltpu.async_copy(ei_hbm.at[w, j], ebuf[s], isem[s])

    def fetch_wait(s):
      pltpu.make_async_copy(ei_hbm.at[w, 0], ebuf[s], isem[s]).wait()

    def scat(s):
      pltpu.async_copy(rones, dacc.at[ebuf[s].at[1]], ssem[s], add=True)

    def scat_wait(s):
      pltpu.make_async_copy(rones, dacc.at[ebuf[s].at[1]], ssem[s]).wait()

    fetch(0, 0)
    fetch(1, 1)
    fetch(2, 2)
    fetch(3, 3)
    fetch_wait(0)
    scat(0)
    fetch_wait(1)
    scat(1)

    def quad(i, carry):
      j0 = 4 * i + 2
      for t in range(4):
        j = j0 + t
        s = (2 + t) % 4
        nxt = (s + 2) % 4
        scat_wait(nxt)
        fetch(j + 2, nxt)
        fetch_wait(s)
        scat(s)
      return carry

    lax.fori_loop(0, (NCH - 5) // 4, quad, 0)

    scat_wait(0)
    fetch(NCH - 1, 0)
    fetch_wait(2)
    scat(2)
    scat_wait(1)
    fetch_wait(3)
    scat(3)
    fetch_wait(0)
    scat(0)
    scat_wait(2)
    scat_wait(3)
    scat_wait(0)

    plsc.subcore_barrier()
    _sc_write_stripe(sid, cid, dacc, deg_out, off)

  return pl.kernel(
      body,
      out_type=jax.ShapeDtypeStruct((NC, N, D), jnp.float32),
      mesh=mesh,
      scratch_types=(
          pltpu.VMEM_SHARED((N, D), jnp.float32),
          pltpu.VMEM((2, CH), jnp.int32),
          pltpu.VMEM((2, CH), jnp.int32),
          pltpu.VMEM((2, CH), jnp.int32),
          pltpu.VMEM((2, CH), jnp.int32),
          pltpu.VMEM((CH, D), jnp.float32),
      ) + (pltpu.SemaphoreType.DMA,) * 9)


BN = 1000
GRID = N // BN


def _tc_pre_body(x_ref, p_ref, deg_ref, wdst_ref, wsrc_ref, wut_ref, wub_ref,
                 bdst_ref, bsrc_ref, bupd_ref, pre_ref, stats_ref):
  s = p_ref[0] + p_ref[1]
  d = deg_ref[0, :, 0:1] + deg_ref[1, :, 0:1]
  mean = s * (1.0 / jnp.maximum(d, 1.0))
  h_dst = jnp.dot(x_ref[...], wdst_ref[...], preferred_element_type=jnp.float32)
  h_src = jnp.dot(mean, wsrc_ref[...], preferred_element_type=jnp.float32)
  bias = (jnp.dot(bdst_ref[...], wut_ref[...], preferred_element_type=jnp.float32)
          + jnp.dot(bsrc_ref[...], wub_ref[...], preferred_element_type=jnp.float32)
          + bupd_ref[...])
  pre = (jnp.dot(h_dst, wut_ref[...], preferred_element_type=jnp.float32)
         + jnp.dot(h_src, wub_ref[...], preferred_element_type=jnp.float32)
         + bias)
  pre_ref[...] = pre

  @pl.when(pl.program_id(0) == 0)
  def _():
    stats_ref[...] = jnp.zeros_like(stats_ref)

  stats_ref[0:1, :] += jnp.sum(pre, axis=0, keepdims=True)
  stats_ref[1:2, :] += jnp.sum(pre * pre, axis=0, keepdims=True)


def _tc_pre(x, parts, degs, w_dst, w_src, w_upd, b_dst, b_src, b_upd):
  wut = w_upd[:H]
  wub = w_upd[H:]
  return pl.pallas_call(
      _tc_pre_body,
      grid=(GRID,),
      in_specs=[
          pl.BlockSpec((BN, D), lambda i: (i, 0)),
          pl.BlockSpec((NC, BN, H), lambda i: (0, i, 0)),
          pl.BlockSpec((NC, BN, D), lambda i: (0, i, 0)),
          pl.BlockSpec((D, H), lambda i: (0, 0)),
          pl.BlockSpec((H, H), lambda i: (0, 0)),
          pl.BlockSpec((H, H), lambda i: (0, 0)),
          pl.BlockSpec((H, H), lambda i: (0, 0)),
          pl.BlockSpec((1, H), lambda i: (0, 0)),
          pl.BlockSpec((1, H), lambda i: (0, 0)),
          pl.BlockSpec((1, H), lambda i: (0, 0)),
      ],
      out_specs=[
          pl.BlockSpec((BN, H), lambda i: (i, 0)),
          pl.BlockSpec((8, H), lambda i: (0, 0)),
      ],
      out_shape=[
          jax.ShapeDtypeStruct((N, H), jnp.float32),
          jax.ShapeDtypeStruct((8, H), jnp.float32),
      ],
  )(x, parts, degs, w_dst, w_src, wut, wub,
    b_dst.reshape(1, H), b_src.reshape(1, H), b_upd.reshape(1, H))


def _tc_norm_body(pre_ref, stats_ref, g_ref, b_ref, out_ref):
  m = stats_ref[0:1, :] / N
  v = stats_ref[1:2, :] / N - m * m
  scale = g_ref[...] / jnp.sqrt(v + 1.0)
  y = (pre_ref[...] - m) * scale + b_ref[...]
  out_ref[...] = jnp.where(y >= 0, y, 0.01 * y)


def _tc_norm(pre, stats, gamma, beta):
  return pl.pallas_call(
      _tc_norm_body,
      grid=(GRID,),
      in_specs=[
          pl.BlockSpec((BN, H), lambda i: (i, 0)),
          pl.BlockSpec((8, H), lambda i: (0, 0)),
          pl.BlockSpec((1, H), lambda i: (0, 0)),
          pl.BlockSpec((1, H), lambda i: (0, 0)),
      ],
      out_specs=pl.BlockSpec((BN, H), lambda i: (i, 0)),
      out_shape=jax.ShapeDtypeStruct((N, H), jnp.float32),
  )(pre, stats, gamma.reshape(1, H), beta.reshape(1, H))


def _tc_norm_head_body(pre_ref, stats_ref, g_ref, b_ref, wp_ref, bp_ref,
                       out_ref):
  m = stats_ref[0:1, :] / N
  v = stats_ref[1:2, :] / N - m * m
  scale = g_ref[...] / jnp.sqrt(v + 1.0)
  y = (pre_ref[...] - m) * scale + b_ref[...]
  y = jnp.where(y >= 0, y, 0.01 * y)
  out_ref[...] = (jnp.dot(y, wp_ref[...], preferred_element_type=jnp.float32)
                  + bp_ref[...])


def _tc_norm_head(pre, stats, gamma, beta, w_post, b_post):
  return pl.pallas_call(
      _tc_norm_head_body,
      grid=(GRID,),
      in_specs=[
          pl.BlockSpec((BN, H), lambda i: (i, 0)),
          pl.BlockSpec((8, H), lambda i: (0, 0)),
          pl.BlockSpec((1, H), lambda i: (0, 0)),
          pl.BlockSpec((1, H), lambda i: (0, 0)),
          pl.BlockSpec((H, L), lambda i: (0, 0)),
          pl.BlockSpec((1, L), lambda i: (0, 0)),
      ],
      out_specs=pl.BlockSpec((BN, L), lambda i: (i, 0)),
      out_shape=jax.ShapeDtypeStruct((N, L), jnp.float32),
  )(pre, stats, gamma.reshape(1, H), beta.reshape(1, H), w_post,
    b_post.reshape(1, L))


@jax.jit
def kernel(node_feature, edge_index, W_src1, b_src1, W_dst1, b_dst1, W_upd1,
           b_upd1, W_src2, b_src2, W_dst2, b_dst2, W_upd2, b_upd2,
           gamma1, beta1, gamma2, beta2, W_post, b_post):
  ei = (edge_index.astype(jnp.int32)
        .reshape(2, NW, NCH, CH).transpose(1, 2, 0, 3))
  zf = jnp.zeros((STRIPE, D), jnp.float32)
  ones = jnp.ones((CH, D), jnp.float32)

  degs = _make_sc_degree()(ei, zf, ones)
  parts1 = _make_sc_aggregate()(node_feature, ei, zf)
  pre1, stats1 = _tc_pre(node_feature, parts1, degs, W_dst1, W_src1, W_upd1,
                         b_dst1, b_src1, b_upd1)
  x1 = _tc_norm(pre1, stats1, gamma1, beta1)

  parts2 = _make_sc_aggregate()(x1, ei, zf)
  pre2, stats2 = _tc_pre(x1, parts2, degs, W_dst2, W_src2, W_upd2,
                         b_dst2, b_src2, b_upd2)
  return _tc_norm_head(pre2, stats2, gamma2, beta2, W_post, b_post)


# speedup vs baseline: 10.9656x; 1.0242x over previous
import functools
import jax
import jax.numpy as jnp
from jax import lax
from jax.experimental import pallas as pl
from jax.experimental.pallas import tpu as pltpu
from jax.experimental.pallas import tpu_sc as plsc

N = 10000
E = 320000
D = 128
H = 128
L = 16

NC = 2
NS = 16
NW = NC * NS
PT = E // NW
CH = 80
DW = 128
NCH = PT // CH
STRIPE = 632
LAST = N - STRIPE * (NS - 1)


def _sc_zero_stripe(sid, ref, zf_hbm, off):
  @pl.when(sid < NS - 1)
  def _():
    pltpu.sync_copy(zf_hbm, ref.at[pl.ds(off, STRIPE)])

  @pl.when(sid == NS - 1)
  def _():
    pltpu.sync_copy(zf_hbm.at[pl.ds(0, LAST)], ref.at[pl.ds(off, LAST)])


def _sc_write_stripe(sid, cid, ref, out, off):
  @pl.when(sid < NS - 1)
  def _():
    pltpu.sync_copy(ref.at[pl.ds(off, STRIPE)], out.at[cid, pl.ds(off, STRIPE)])

  @pl.when(sid == NS - 1)
  def _():
    pltpu.sync_copy(ref.at[pl.ds(off, LAST)], out.at[cid, pl.ds(off, LAST)])


@functools.lru_cache(None)
def _make_sc_aggregate():
  mesh = plsc.VectorSubcoreMesh(core_axis_name="c", subcore_axis_name="s")

  def body(x_hbm, ei_hbm, zf_hbm, feat_out, acc,
           e0, e1, e2, e3, r0, r1, r2, r3,
           is0, is1, is2, is3, gs0, gs1, gs2, gs3, ss0, ss1, ss2, ss3):
    ebuf = (e0, e1, e2, e3)
    rbuf = (r0, r1, r2, r3)
    isem = (is0, is1, is2, is3)
    gsem = (gs0, gs1, gs2, gs3)
    ssem = (ss0, ss1, ss2, ss3)
    cid = lax.axis_index("c")
    sid = lax.axis_index("s")
    w = cid * NS + sid
    off = pl.multiple_of(sid * STRIPE, 8)

    _sc_zero_stripe(sid, acc, zf_hbm, off)
    plsc.subcore_barrier()

    def fetch(j, s):
      pltpu.async_copy(ei_hbm.at[w, j], ebuf[s], isem[s])

    def fetch_wait(s):
      pltpu.make_async_copy(ei_hbm.at[w, 0], ebuf[s], isem[s]).wait()

    def gather(s):
      pltpu.async_copy(x_hbm.at[ebuf[s].at[0]], rbuf[s], gsem[s])

    def gather_wait(s):
      pltpu.make_async_copy(x_hbm.at[ebuf[s].at[0]], rbuf[s], gsem[s]).wait()

    def scat(s):
      pltpu.async_copy(rbuf[s], acc.at[ebuf[s].at[1]], ssem[s], add=True)

    def scat_wait(s):
      pltpu.make_async_copy(rbuf[s], acc.at[ebuf[s].at[1]], ssem[s]).wait()

    fetch(0, 0)
    fetch(1, 1)
    fetch_wait(0)
    gather(0)
    fetch_wait(1)
    gather(1)
    fetch(2, 2)
    fetch_wait(2)
    gather(2)
    gather_wait(0)
    scat(0)
    fetch(3, 3)
    fetch_wait(3)
    gather(3)
    gather_wait(1)
    scat(1)

    def quad(i, carry):
      j0 = 4 * i + 2
      for t in range(4):
        j = j0 + t
        s = (2 + t) % 4
        nxt = (s + 2) % 4
        scat_wait(nxt)
        fetch(j + 2, nxt)
        fetch_wait(nxt)
        gather(nxt)
        gather_wait(s)
        scat(s)
      return carry

    lax.fori_loop(0, (NCH - 5) // 4, quad, 0)

    scat_wait(0)
    fetch(NCH - 1, 0)
    fetch_wait(0)
    gather(0)
    gather_wait(2)
    scat(2)
    scat_wait(1)
    gather_wait(3)
    scat(3)
    gather_wait(0)
    scat(0)
    scat_wait(2)
    scat_wait(3)
    scat_wait(0)

    plsc.subcore_barrier()
    _sc_write_stripe(sid, cid, acc, feat_out, off)

  return pl.kernel(
      body,
      out_type=jax.ShapeDtypeStruct((NC, N, D), jnp.float32),
      mesh=mesh,
      scratch_types=(
          pltpu.VMEM_SHARED((N, D), jnp.float32),
          pltpu.VMEM((2, CH), jnp.int32),
          pltpu.VMEM((2, CH), jnp.int32),
          pltpu.VMEM((2, CH), jnp.int32),
          pltpu.VMEM((2, CH), jnp.int32),
          pltpu.VMEM((CH, D), jnp.float32),
          pltpu.VMEM((CH, D), jnp.float32),
          pltpu.VMEM((CH, D), jnp.float32),
          pltpu.VMEM((CH, D), jnp.float32),
      ) + (pltpu.SemaphoreType.DMA,) * 12)


@functools.lru_cache(None)
def _make_sc_degree():
  mesh = plsc.VectorSubcoreMesh(core_axis_name="c", subcore_axis_name="s")

  def body(ei_hbm, zd_hbm, ones_hbm, deg_out,
           dacc, e0, e1, e2, e3, rones,
           is0, is1, is2, is3, ss0, ss1, ss2, ss3, osem):
    ebuf = (e0, e1, e2, e3)
    isem = (is0, is1, is2, is3)
    ssem = (ss0, ss1, ss2, ss3)
    cid = lax.axis_index("c")
    sid = lax.axis_index("s")
    w = cid * NS + sid
    off = pl.multiple_of(sid * STRIPE, 8)

    _sc_zero_stripe(sid, dacc, zd_hbm, off)
    pltpu.async_copy(ones_hbm, rones, osem).wait()
    plsc.subcore_barrier()

    def fetch(j, s):
      pltpu.async_copy(ei_hbm.at[w, j], ebuf[s], isem[s])

    def fetch_wait(s):
      pltpu.make_async_copy(ei_hbm.at[w, 0], ebuf[s], isem[s]).wait()

    def scat(s):
      pltpu.async_copy(rones, dacc.at[ebuf[s].at[1]], ssem[s], add=True)

    def scat_wait(s):
      pltpu.make_async_copy(rones, dacc.at[ebuf[s].at[1]], ssem[s]).wait()

    fetch(0, 0)
    fetch(1, 1)
    fetch(2, 2)
    fetch(3, 3)
    fetch_wait(0)
    scat(0)
    fetch_wait(1)
    scat(1)

    def quad(i, carry):
      j0 = 4 * i + 2
      for t in range(4):
        j = j0 + t
        s = (2 + t) % 4
        nxt = (s + 2) % 4
        scat_wait(nxt)
        fetch(j + 2, nxt)
        fetch_wait(s)
        scat(s)
      return carry

    lax.fori_loop(0, (NCH - 5) // 4, quad, 0)

    scat_wait(0)
    fetch(NCH - 1, 0)
    fetch_wait(2)
    scat(2)
    scat_wait(1)
    fetch_wait(3)
    scat(3)
    fetch_wait(0)
    scat(0)
    scat_wait(2)
    scat_wait(3)
    scat_wait(0)

    plsc.subcore_barrier()
    _sc_write_stripe(sid, cid, dacc, deg_out, off)

  return pl.kernel(
      body,
      out_type=jax.ShapeDtypeStruct((NC, N, DW), jnp.float32),
      mesh=mesh,
      scratch_types=(
          pltpu.VMEM_SHARED((N, DW), jnp.float32),
          pltpu.VMEM((2, CH), jnp.int32),
          pltpu.VMEM((2, CH), jnp.int32),
          pltpu.VMEM((2, CH), jnp.int32),
          pltpu.VMEM((2, CH), jnp.int32),
          pltpu.VMEM((CH, DW), jnp.float32),
      ) + (pltpu.SemaphoreType.DMA,) * 9)


BN = 1000
GRID = N // BN


def _bn_scale(stats_s, g_ref):
  m = stats_s[0:1, :] / N
  v = stats_s[1:2, :] / N - m * m
  return m, g_ref[...] / jnp.sqrt(v + 1.0)


def _tc_layer1_body(x_ref, p_ref, deg_ref, wdst_ref, wsrc_ref, wut_ref,
                    wub_ref, bdst_ref, bsrc_ref, bupd_ref, g_ref, b_ref,
                    x1_ref, recip_ref, pre_s, stats_s, recip_s):
  ph = pl.program_id(0)
  i = pl.program_id(1)

  @pl.when(ph == 0)
  def _():
    d = deg_ref[0, :, 0:1] + deg_ref[1, :, 0:1]
    r = 1.0 / jnp.maximum(d, 1.0)
    recip_s[pl.ds(i * BN, BN), :] = r
    mean = (p_ref[0] + p_ref[1]) * r
    h_dst = jnp.dot(x_ref[...], wdst_ref[...],
                    preferred_element_type=jnp.float32)
    h_src = jnp.dot(mean, wsrc_ref[...], preferred_element_type=jnp.float32)
    bias = (jnp.dot(bdst_ref[...], wut_ref[...],
                    preferred_element_type=jnp.float32)
            + jnp.dot(bsrc_ref[...], wub_ref[...],
                      preferred_element_type=jnp.float32)
            + bupd_ref[...])
    pre = (jnp.dot(h_dst, wut_ref[...], preferred_element_type=jnp.float32)
           + jnp.dot(h_src, wub_ref[...], preferred_element_type=jnp.float32)
           + bias)
    pre_s[pl.ds(i * BN, BN), :] = pre

    @pl.when(i == 0)
    def _():
      stats_s[...] = jnp.zeros_like(stats_s)

    stats_s[0:1, :] += jnp.sum(pre, axis=0, keepdims=True)
    stats_s[1:2, :] += jnp.sum(pre * pre, axis=0, keepdims=True)

  @pl.when(ph == 1)
  def _():
    m, scale = _bn_scale(stats_s, g_ref)
    y = (pre_s[pl.ds(i * BN, BN), :] - m) * scale + b_ref[...]
    x1_ref[...] = jnp.where(y >= 0, y, 0.01 * y)
    recip_ref[...] = recip_s[pl.ds(i * BN, BN), :]


def _tc_layer1(x, parts, degs, w_dst, w_src, w_upd, b_dst, b_src, b_upd,
               gamma, beta):
  return pl.pallas_call(
      _tc_layer1_body,
      grid=(2, GRID),
      in_specs=[
          pl.BlockSpec((BN, D), lambda p, i: (i * (1 - p), 0)),
          pl.BlockSpec((NC, BN, H), lambda p, i: (0, i * (1 - p), 0)),
          pl.BlockSpec((NC, BN, DW), lambda p, i: (0, i * (1 - p), 0)),
          pl.BlockSpec((D, H), lambda p, i: (0, 0)),
          pl.BlockSpec((H, H), lambda p, i: (0, 0)),
          pl.BlockSpec((H, H), lambda p, i: (0, 0)),
          pl.BlockSpec((H, H), lambda p, i: (0, 0)),
          pl.BlockSpec((1, H), lambda p, i: (0, 0)),
          pl.BlockSpec((1, H), lambda p, i: (0, 0)),
          pl.BlockSpec((1, H), lambda p, i: (0, 0)),
          pl.BlockSpec((1, H), lambda p, i: (0, 0)),
          pl.BlockSpec((1, H), lambda p, i: (0, 0)),
      ],
      out_specs=[
          pl.BlockSpec((BN, H), lambda p, i: (i * p, 0)),
          pl.BlockSpec((BN, 1), lambda p, i: (i * p, 0)),
      ],
      out_shape=[
          jax.ShapeDtypeStruct((N, H), jnp.float32),
          jax.ShapeDtypeStruct((N, 1), jnp.float32),
      ],
      scratch_shapes=[
          pltpu.VMEM((N, H), jnp.float32),
          pltpu.VMEM((8, H), jnp.float32),
          pltpu.VMEM((N, 1), jnp.float32),
      ],
  )(x, parts, degs, w_dst, w_src, w_upd[:H], w_upd[H:],
    b_dst.reshape(1, H), b_src.reshape(1, H), b_upd.reshape(1, H),
    gamma.reshape(1, H), beta.reshape(1, H))


def _tc_layer2_body(x_ref, p_ref, recip_ref, wdst_ref, wsrc_ref, wut_ref,
                    wub_ref, bdst_ref, bsrc_ref, bupd_ref, g_ref, b_ref,
                    wp_ref, bp_ref, out_ref, pre_s, stats_s):
  ph = pl.program_id(0)
  i = pl.program_id(1)

  @pl.when(ph == 0)
  def _():
    mean = (p_ref[0] + p_ref[1]) * recip_ref[...]
    h_dst = jnp.dot(x_ref[...], wdst_ref[...],
                    preferred_element_type=jnp.float32)
    h_src = jnp.dot(mean, wsrc_ref[...], preferred_element_type=jnp.float32)
    bias = (jnp.dot(bdst_ref[...], wut_ref[...],
                    preferred_element_type=jnp.float32)
            + jnp.dot(bsrc_ref[...], wub_ref[...],
                      preferred_element_type=jnp.float32)
            + bupd_ref[...])
    pre = (jnp.dot(h_dst, wut_ref[...], preferred_element_type=jnp.float32)
           + jnp.dot(h_src, wub_ref[...], preferred_element_type=jnp.float32)
           + bias)
    pre_s[pl.ds(i * BN, BN), :] = pre

    @pl.when(i == 0)
    def _():
      stats_s[...] = jnp.zeros_like(stats_s)

    stats_s[0:1, :] += jnp.sum(pre, axis=0, keepdims=True)
    stats_s[1:2, :] += jnp.sum(pre * pre, axis=0, keepdims=True)

  @pl.when(ph == 1)
  def _():
    m, scale = _bn_scale(stats_s, g_ref)
    y = (pre_s[pl.ds(i * BN, BN), :] - m) * scale + b_ref[...]
    y = jnp.where(y >= 0, y, 0.01 * y)
    out_ref[...] = (jnp.dot(y, wp_ref[...], preferred_element_type=jnp.float32)
                    + bp_ref[...])


def _tc_layer2(x, parts, recip, w_dst, w_src, w_upd, b_dst, b_src, b_upd,
               gamma, beta, w_post, b_post):
  return pl.pallas_call(
      _tc_layer2_body,
      grid=(2, GRID),
      in_specs=[
          pl.BlockSpec((BN, H), lambda p, i: (i * (1 - p), 0)),
          pl.BlockSpec((NC, BN, H), lambda p, i: (0, i * (1 - p), 0)),
          pl.BlockSpec((BN, 1), lambda p, i: (i * (1 - p), 0)),
          pl.BlockSpec((H, H), lambda p, i: (0, 0)),
          pl.BlockSpec((H, H), lambda p, i: (0, 0)),
          pl.BlockSpec((H, H), lambda p, i: (0, 0)),
          pl.BlockSpec((H, H), lambda p, i: (0, 0)),
          pl.BlockSpec((1, H), lambda p, i: (0, 0)),
          pl.BlockSpec((1, H), lambda p, i: (0, 0)),
          pl.BlockSpec((1, H), lambda p, i: (0, 0)),
          pl.BlockSpec((1, H), lambda p, i: (0, 0)),
          pl.BlockSpec((1, H), lambda p, i: (0, 0)),
          pl.BlockSpec((H, L), lambda p, i: (0, 0)),
          pl.BlockSpec((1, L), lambda p, i: (0, 0)),
      ],
      out_specs=pl.BlockSpec((BN, L), lambda p, i: (i * p, 0)),
      out_shape=jax.ShapeDtypeStruct((N, L), jnp.float32),
      scratch_shapes=[
          pltpu.VMEM((N, H), jnp.float32),
          pltpu.VMEM((8, H), jnp.float32),
      ],
  )(x, parts, recip, w_dst, w_src, w_upd[:H], w_upd[H:],
    b_dst.reshape(1, H), b_src.reshape(1, H), b_upd.reshape(1, H),
    gamma.reshape(1, H), beta.reshape(1, H), w_post, b_post.reshape(1, L))


@jax.jit
def kernel(node_feature, edge_index, W_src1, b_src1, W_dst1, b_dst1, W_upd1,
           b_upd1, W_src2, b_src2, W_dst2, b_dst2, W_upd2, b_upd2,
           gamma1, beta1, gamma2, beta2, W_post, b_post):
  ei = (edge_index.astype(jnp.int32)
        .reshape(2, NW, NCH, CH).transpose(1, 2, 0, 3))
  zf = jnp.zeros((STRIPE, D), jnp.float32)
  zd = jnp.zeros((STRIPE, DW), jnp.float32)
  ones = jnp.ones((CH, DW), jnp.float32)

  degs = _make_sc_degree()(ei, zd, ones)
  parts1 = _make_sc_aggregate()(node_feature, ei, zf)
  x1, recip = _tc_layer1(node_feature, parts1, degs, W_dst1, W_src1, W_upd1,
                         b_dst1, b_src1, b_upd1, gamma1, beta1)
  parts2 = _make_sc_aggregate()(x1, ei, zf)
  return _tc_layer2(x1, parts2, recip, W_dst2, W_src2, W_upd2,
                    b_dst2, b_src2, b_upd2, gamma2, beta2, W_post, b_post)


# speedup vs baseline: 11.2170x; 1.0229x over previous
import functools
import jax
import jax.numpy as jnp
from jax import lax
from jax.experimental import pallas as pl
from jax.experimental.pallas import tpu as pltpu
from jax.experimental.pallas import tpu_sc as plsc

N = 10000
E = 320000
D = 128
H = 128
L = 16

NC = 2
NS = 16
NW = NC * NS
PT = E // NW
CH = 80
DW = 128
NCH = PT // CH
STRIPE = 632
LAST = N - STRIPE * (NS - 1)


def _sc_zero_stripe(sid, ref, zf_hbm, off):
  @pl.when(sid < NS - 1)
  def _():
    pltpu.sync_copy(zf_hbm, ref.at[pl.ds(off, STRIPE)])

  @pl.when(sid == NS - 1)
  def _():
    pltpu.sync_copy(zf_hbm.at[pl.ds(0, LAST)], ref.at[pl.ds(off, LAST)])


def _sc_write_stripe(sid, cid, ref, out, off):
  @pl.when(sid < NS - 1)
  def _():
    pltpu.sync_copy(ref.at[pl.ds(off, STRIPE)], out.at[cid, pl.ds(off, STRIPE)])

  @pl.when(sid == NS - 1)
  def _():
    pltpu.sync_copy(ref.at[pl.ds(off, LAST)], out.at[cid, pl.ds(off, LAST)])


@functools.lru_cache(None)
def _make_sc_aggregate(with_deg: bool):
  mesh = plsc.VectorSubcoreMesh(core_axis_name="c", subcore_axis_name="s")

  def body(x_hbm, ei_hbm, zf_hbm, ones_hbm, *rest):
    if with_deg:
      feat_out, deg_out = rest[0], rest[1]
      rest = rest[2:]
    else:
      feat_out = rest[0]
      rest = rest[1:]
    (acc, e0, e1, e2, e3, r0, r1, r2, r3,
     is0, is1, is2, is3, gs0, gs1, gs2, gs3, ss0, ss1, ss2, ss3) = rest
    ebuf = (e0, e1, e2, e3)
    rbuf = (r0, r1, r2, r3)
    isem = (is0, is1, is2, is3)
    gsem = (gs0, gs1, gs2, gs3)
    ssem = (ss0, ss1, ss2, ss3)
    cid = lax.axis_index("c")
    sid = lax.axis_index("s")
    w = cid * NS + sid
    off = pl.multiple_of(sid * STRIPE, 8)

    _sc_zero_stripe(sid, acc, zf_hbm, off)
    if with_deg:
      pltpu.async_copy(ones_hbm, r0, gs0).wait()
      plsc.subcore_barrier()

      def dfetch(j, s):
        pltpu.async_copy(ei_hbm.at[w, j], ebuf[s], isem[s])

      def dfetch_wait(s):
        pltpu.make_async_copy(ei_hbm.at[w, 0], ebuf[s], isem[s]).wait()

      def dscat(s):
        pltpu.async_copy(r0, acc.at[ebuf[s].at[1]], ssem[s], add=True)

      def dscat_wait(s):
        pltpu.make_async_copy(r0, acc.at[ebuf[s].at[1]], ssem[s]).wait()

      dfetch(0, 0)
      dfetch(1, 1)
      dfetch(2, 2)
      dfetch(3, 3)
      dfetch_wait(0)
      dscat(0)
      dfetch_wait(1)
      dscat(1)

      def dquad(i, carry):
        j0 = 4 * i + 2
        for t in range(4):
          j = j0 + t
          s = (2 + t) % 4
          nxt = (s + 2) % 4
          dscat_wait(nxt)
          dfetch(j + 2, nxt)
          dfetch_wait(s)
          dscat(s)
        return carry

      lax.fori_loop(0, (NCH - 5) // 4, dquad, 0)
      dscat_wait(0)
      dfetch(NCH - 1, 0)
      dfetch_wait(2)
      dscat(2)
      dscat_wait(1)
      dfetch_wait(3)
      dscat(3)
      dfetch_wait(0)
      dscat(0)
      dscat_wait(2)
      dscat_wait(3)
      dscat_wait(0)

      plsc.subcore_barrier()
      _sc_write_stripe(sid, cid, acc, deg_out, off)
      _sc_zero_stripe(sid, acc, zf_hbm, off)
    plsc.subcore_barrier()

    def fetch(j, s):
      pltpu.async_copy(ei_hbm.at[w, j], ebuf[s], isem[s])

    def fetch_wait(s):
      pltpu.make_async_copy(ei_hbm.at[w, 0], ebuf[s], isem[s]).wait()

    def gather(s):
      pltpu.async_copy(x_hbm.at[ebuf[s].at[0]], rbuf[s], gsem[s])

    def gather_wait(s):
      pltpu.make_async_copy(x_hbm.at[ebuf[s].at[0]], rbuf[s], gsem[s]).wait()

    def scat(s):
      pltpu.async_copy(rbuf[s], acc.at[ebuf[s].at[1]], ssem[s], add=True)

    def scat_wait(s):
      pltpu.make_async_copy(rbuf[s], acc.at[ebuf[s].at[1]], ssem[s]).wait()

    fetch(0, 0)
    fetch(1, 1)
    fetch_wait(0)
    gather(0)
    fetch_wait(1)
    gather(1)
    fetch(2, 2)
    fetch_wait(2)
    gather(2)
    gather_wait(0)
    scat(0)
    fetch(3, 3)
    fetch_wait(3)
    gather(3)
    gather_wait(1)
    scat(1)

    def quad(i, carry):
      j0 = 4 * i + 2
      for t in range(4):
        j = j0 + t
        s = (2 + t) % 4
        nxt = (s + 2) % 4
        scat_wait(nxt)
        fetch(j + 2, nxt)
        fetch_wait(nxt)
        gather(nxt)
        gather_wait(s)
        scat(s)
      return carry

    lax.fori_loop(0, (NCH - 5) // 4, quad, 0)

    scat_wait(0)
    fetch(NCH - 1, 0)
    fetch_wait(0)
    gather(0)
    gather_wait(2)
    scat(2)
    scat_wait(1)
    gather_wait(3)
    scat(3)
    gather_wait(0)
    scat(0)
    scat_wait(2)
    scat_wait(3)
    scat_wait(0)

    plsc.subcore_barrier()
    _sc_write_stripe(sid, cid, acc, feat_out, off)

  out_type = [jax.ShapeDtypeStruct((NC, N, D), jnp.float32)]
  if with_deg:
    out_type.append(jax.ShapeDtypeStruct((NC, N, DW), jnp.float32))
  return pl.kernel(
      body,
      out_type=tuple(out_type),
      mesh=mesh,
      scratch_types=(
          pltpu.VMEM_SHARED((N, D), jnp.float32),
          pltpu.VMEM((2, CH), jnp.int32),
          pltpu.VMEM((2, CH), jnp.int32),
          pltpu.VMEM((2, CH), jnp.int32),
          pltpu.VMEM((2, CH), jnp.int32),
          pltpu.VMEM((CH, D), jnp.float32),
          pltpu.VMEM((CH, D), jnp.float32),
          pltpu.VMEM((CH, D), jnp.float32),
          pltpu.VMEM((CH, D), jnp.float32),
      ) + (pltpu.SemaphoreType.DMA,) * 12)


BN = 1000
GRID = N // BN


def _bn_scale(stats_s, g_ref):
  m = stats_s[0:1, :] / N
  v = stats_s[1:2, :] / N - m * m
  return m, g_ref[...] / jnp.sqrt(v + 1.0)


def _tc_layer1_body(x_ref, p_ref, deg_ref, wdst_ref, wsrc_ref, wut_ref,
                    wub_ref, bdst_ref, bsrc_ref, bupd_ref, g_ref, b_ref,
                    x1_ref, recip_ref, pre_s, stats_s, recip_s):
  ph = pl.program_id(0)
  i = pl.program_id(1)

  @pl.when(ph == 0)
  def _():
    d = deg_ref[0, :, 0:1] + deg_ref[1, :, 0:1]
    r = 1.0 / jnp.maximum(d, 1.0)
    recip_s[pl.ds(i * BN, BN), :] = r
    mean = (p_ref[0] + p_ref[1]) * r
    h_dst = jnp.dot(x_ref[...], wdst_ref[...],
                    preferred_element_type=jnp.float32)
    h_src = jnp.dot(mean, wsrc_ref[...], preferred_element_type=jnp.float32)
    bias = (jnp.dot(bdst_ref[...], wut_ref[...],
                    preferred_element_type=jnp.float32)
            + jnp.dot(bsrc_ref[...], wub_ref[...],
                      preferred_element_type=jnp.float32)
            + bupd_ref[...])
    pre = (jnp.dot(h_dst, wut_ref[...], preferred_element_type=jnp.float32)
           + jnp.dot(h_src, wub_ref[...], preferred_element_type=jnp.float32)
           + bias)
    pre_s[pl.ds(i * BN, BN), :] = pre

    @pl.when(i == 0)
    def _():
      stats_s[...] = jnp.zeros_like(stats_s)

    stats_s[0:1, :] += jnp.sum(pre, axis=0, keepdims=True)
    stats_s[1:2, :] += jnp.sum(pre * pre, axis=0, keepdims=True)

  @pl.when(ph == 1)
  def _():
    m, scale = _bn_scale(stats_s, g_ref)
    y = (pre_s[pl.ds(i * BN, BN), :] - m) * scale + b_ref[...]
    x1_ref[...] = jnp.where(y >= 0, y, 0.01 * y)
    recip_ref[...] = recip_s[pl.ds(i * BN, BN), :]


def _tc_layer1(x, parts, degs, w_dst, w_src, w_upd, b_dst, b_src, b_upd,
               gamma, beta):
  return pl.pallas_call(
      _tc_layer1_body,
      grid=(2, GRID),
      in_specs=[
          pl.BlockSpec((BN, D), lambda p, i: (i * (1 - p), 0)),
          pl.BlockSpec((NC, BN, H), lambda p, i: (0, i * (1 - p), 0)),
          pl.BlockSpec((NC, BN, DW), lambda p, i: (0, i * (1 - p), 0)),
          pl.BlockSpec((D, H), lambda p, i: (0, 0)),
          pl.BlockSpec((H, H), lambda p, i: (0, 0)),
          pl.BlockSpec((H, H), lambda p, i: (0, 0)),
          pl.BlockSpec((H, H), lambda p, i: (0, 0)),
          pl.BlockSpec((1, H), lambda p, i: (0, 0)),
          pl.BlockSpec((1, H), lambda p, i: (0, 0)),
          pl.BlockSpec((1, H), lambda p, i: (0, 0)),
          pl.BlockSpec((1, H), lambda p, i: (0, 0)),
          pl.BlockSpec((1, H), lambda p, i: (0, 0)),
      ],
      out_specs=[
          pl.BlockSpec((BN, H), lambda p, i: (i * p, 0)),
          pl.BlockSpec((BN, 1), lambda p, i: (i * p, 0)),
      ],
      out_shape=[
          jax.ShapeDtypeStruct((N, H), jnp.float32),
          jax.ShapeDtypeStruct((N, 1), jnp.float32),
      ],
      scratch_shapes=[
          pltpu.VMEM((N, H), jnp.float32),
          pltpu.VMEM((8, H), jnp.float32),
          pltpu.VMEM((N, 1), jnp.float32),
      ],
  )(x, parts, degs, w_dst, w_src, w_upd[:H], w_upd[H:],
    b_dst.reshape(1, H), b_src.reshape(1, H), b_upd.reshape(1, H),
    gamma.reshape(1, H), beta.reshape(1, H))


def _tc_layer2_body(x_ref, p_ref, recip_ref, wdst_ref, wsrc_ref, wut_ref,
                    wub_ref, bdst_ref, bsrc_ref, bupd_ref, g_ref, b_ref,
                    wp_ref, bp_ref, out_ref, pre_s, stats_s):
  ph = pl.program_id(0)
  i = pl.program_id(1)

  @pl.when(ph == 0)
  def _():
    mean = (p_ref[0] + p_ref[1]) * recip_ref[...]
    h_dst = jnp.dot(x_ref[...], wdst_ref[...],
                    preferred_element_type=jnp.float32)
    h_src = jnp.dot(mean, wsrc_ref[...], preferred_element_type=jnp.float32)
    bias = (jnp.dot(bdst_ref[...], wut_ref[...],
                    preferred_element_type=jnp.float32)
            + jnp.dot(bsrc_ref[...], wub_ref[...],
                      preferred_element_type=jnp.float32)
            + bupd_ref[...])
    pre = (jnp.dot(h_dst, wut_ref[...], preferred_element_type=jnp.float32)
           + jnp.dot(h_src, wub_ref[...], preferred_element_type=jnp.float32)
           + bias)
    pre_s[pl.ds(i * BN, BN), :] = pre

    @pl.when(i == 0)
    def _():
      stats_s[...] = jnp.zeros_like(stats_s)

    stats_s[0:1, :] += jnp.sum(pre, axis=0, keepdims=True)
    stats_s[1:2, :] += jnp.sum(pre * pre, axis=0, keepdims=True)

  @pl.when(ph == 1)
  def _():
    m, scale = _bn_scale(stats_s, g_ref)
    y = (pre_s[pl.ds(i * BN, BN), :] - m) * scale + b_ref[...]
    y = jnp.where(y >= 0, y, 0.01 * y)
    out_ref[...] = (jnp.dot(y, wp_ref[...], preferred_element_type=jnp.float32)
                    + bp_ref[...])


def _tc_layer2(x, parts, recip, w_dst, w_src, w_upd, b_dst, b_src, b_upd,
               gamma, beta, w_post, b_post):
  return pl.pallas_call(
      _tc_layer2_body,
      grid=(2, GRID),
      in_specs=[
          pl.BlockSpec((BN, H), lambda p, i: (i * (1 - p), 0)),
          pl.BlockSpec((NC, BN, H), lambda p, i: (0, i * (1 - p), 0)),
          pl.BlockSpec((BN, 1), lambda p, i: (i * (1 - p), 0)),
          pl.BlockSpec((H, H), lambda p, i: (0, 0)),
          pl.BlockSpec((H, H), lambda p, i: (0, 0)),
          pl.BlockSpec((H, H), lambda p, i: (0, 0)),
          pl.BlockSpec((H, H), lambda p, i: (0, 0)),
          pl.BlockSpec((1, H), lambda p, i: (0, 0)),
          pl.BlockSpec((1, H), lambda p, i: (0, 0)),
          pl.BlockSpec((1, H), lambda p, i: (0, 0)),
          pl.BlockSpec((1, H), lambda p, i: (0, 0)),
          pl.BlockSpec((1, H), lambda p, i: (0, 0)),
          pl.BlockSpec((H, L), lambda p, i: (0, 0)),
          pl.BlockSpec((1, L), lambda p, i: (0, 0)),
      ],
      out_specs=pl.BlockSpec((BN, L), lambda p, i: (i * p, 0)),
      out_shape=jax.ShapeDtypeStruct((N, L), jnp.float32),
      scratch_shapes=[
          pltpu.VMEM((N, H), jnp.float32),
          pltpu.VMEM((8, H), jnp.float32),
      ],
  )(x, parts, recip, w_dst, w_src, w_upd[:H], w_upd[H:],
    b_dst.reshape(1, H), b_src.reshape(1, H), b_upd.reshape(1, H),
    gamma.reshape(1, H), beta.reshape(1, H), w_post, b_post.reshape(1, L))


@jax.jit
def kernel(node_feature, edge_index, W_src1, b_src1, W_dst1, b_dst1, W_upd1,
           b_upd1, W_src2, b_src2, W_dst2, b_dst2, W_upd2, b_upd2,
           gamma1, beta1, gamma2, beta2, W_post, b_post):
  ei = (edge_index.astype(jnp.int32)
        .reshape(2, NW, NCH, CH).transpose(1, 2, 0, 3))
  zf = jnp.zeros((STRIPE, D), jnp.float32)
  ones = jnp.ones((CH, D), jnp.float32)

  parts1, degs = jax.tree.leaves(
      _make_sc_aggregate(True)(node_feature, ei, zf, ones))
  x1, recip = _tc_layer1(node_feature, parts1, degs, W_dst1, W_src1, W_upd1,
                         b_dst1, b_src1, b_upd1, gamma1, beta1)
  (parts2,) = jax.tree.leaves(_make_sc_aggregate(False)(x1, ei, zf, ones))
  return _tc_layer2(x1, parts2, recip, W_dst2, W_src2, W_upd2,
                    b_dst2, b_src2, b_upd2, gamma2, beta2, W_post, b_post)
